# Initial kernel scaffold; baseline (speedup 1.0000x reference)
import jax
import jax.numpy as jnp
from jax.experimental import pallas as pl


def kernel(rpn_probs, rpn_bbox, anchors):
    raise NotImplementedError("write your pallas kernel here")


# speedup vs baseline: 4.5618x; 4.5618x over previous
import functools

import jax
import jax.numpy as jnp
import numpy as np
from jax.experimental import pallas as pl

_PROPOSAL_COUNT = 2000
_PRE_NMS = 6000
_K_PAD = 6144
_NMS_THRESHOLD = 0.7
_STD_DEV = np.array([0.1, 0.1, 0.2, 0.2], dtype=np.float32)


def _refine_nms_kernel(a_ref, d_ref, box_ref, keep_ref):
    a = a_ref[0]
    d = d_ref[0]
    ay1, ax1, ay2, ax2 = a[0:1], a[1:2], a[2:3], a[3:4]
    dy, dx, dh, dw = d[0:1], d[1:2], d[2:3], d[3:4]

    h = ay2 - ay1
    w = ax2 - ax1
    cy = ay1 + 0.5 * h + dy * h
    cx = ax1 + 0.5 * w + dx * w
    h = h * jnp.exp(dh)
    w = w * jnp.exp(dw)
    y1 = jnp.clip(cy - 0.5 * h, 0.0, 1.0)
    x1 = jnp.clip(cx - 0.5 * w, 0.0, 1.0)
    y2 = jnp.clip(cy + 0.5 * h, 0.0, 1.0)
    x2 = jnp.clip(cx + 0.5 * w, 0.0, 1.0)
    area = (y2 - y1) * (x2 - x1)

    box_ref[0] = jnp.concatenate([y1, x1, y2, x2], axis=0)

    ar = jax.lax.broadcasted_iota(jnp.int32, (1, _K_PAD), 1)
    keep0 = (ar < _PRE_NMS).astype(jnp.float32)

    def body(i, keepf):
        m = (ar == i).astype(jnp.float32)
        by1 = jnp.sum(y1 * m)
        bx1 = jnp.sum(x1 * m)
        by2 = jnp.sum(y2 * m)
        bx2 = jnp.sum(x2 * m)
        barea = jnp.sum(area * m)
        ki = jnp.sum(keepf * m)
        iy1 = jnp.maximum(by1, y1)
        ix1 = jnp.maximum(bx1, x1)
        iy2 = jnp.minimum(by2, y2)
        ix2 = jnp.minimum(bx2, x2)
        inter = jnp.maximum(iy2 - iy1, 0.0) * jnp.maximum(ix2 - ix1, 0.0)
        union = barea + area - inter
        iou = inter / (union + 1e-8)
        sup = (iou > _NMS_THRESHOLD) & (ki > 0.5) & (ar > i)
        return jnp.where(sup, 0.0, keepf)

    keep_ref[0] = jax.lax.fori_loop(0, _PRE_NMS, body, keep0)


@functools.partial(jax.jit)
def kernel(rpn_probs, rpn_bbox, anchors):
    B = rpn_probs.shape[0]
    scores = rpn_probs[:, :, 1]
    _, ix = jax.lax.top_k(scores, _PRE_NMS)
    d = jnp.take_along_axis(rpn_bbox, ix[..., None], axis=1) * jnp.asarray(
        _STD_DEV
    ).reshape(1, 1, 4)
    a = jnp.take_along_axis(anchors, ix[..., None], axis=1)
    pad = _K_PAD - _PRE_NMS
    d = jnp.pad(d, ((0, 0), (0, pad), (0, 0)))
    a = jnp.pad(a, ((0, 0), (0, pad), (0, 0)))
    a_t = a.transpose(0, 2, 1)
    d_t = d.transpose(0, 2, 1)

    boxes_t, keep = pl.pallas_call(
        _refine_nms_kernel,
        grid=(B,),
        in_specs=[
            pl.BlockSpec((1, 4, _K_PAD), lambda b: (b, 0, 0)),
            pl.BlockSpec((1, 4, _K_PAD), lambda b: (b, 0, 0)),
        ],
        out_specs=[
            pl.BlockSpec((1, 4, _K_PAD), lambda b: (b, 0, 0)),
            pl.BlockSpec((1, 1, _K_PAD), lambda b: (b, 0, 0)),
        ],
        out_shape=[
            jax.ShapeDtypeStruct((B, 4, _K_PAD), jnp.float32),
            jax.ShapeDtypeStruct((B, 1, _K_PAD), jnp.float32),
        ],
    )(a_t, d_t)

    boxes = boxes_t.transpose(0, 2, 1)
    keep_b = keep[:, 0, :] > 0.5

    def compact(kb, bx):
        idx = jnp.nonzero(kb, size=_PROPOSAL_COUNT, fill_value=-1)[0]
        valid = (idx >= 0).astype(bx.dtype)[:, None]
        return jnp.take(bx, jnp.clip(idx, 0, _PRE_NMS - 1), axis=0) * valid

    return jax.vmap(compact)(keep_b, boxes)
